# baseline (speedup 1.0000x reference)
import functools

import jax
import jax.numpy as jnp
from jax import lax
from jax.experimental import pallas as pl
from jax.experimental.pallas import tpu as pltpu
from jax.experimental.pallas import tpu_sc as plsc

VOCAB = 100000
EMB = 300
HID = 128
B = 4096
L = 50

NC = 2
NS = 16
NW = NC * NS
BAGS_PER_W = B // NW
CHUNK_BAGS = 2
ROWS = CHUNK_BAGS * L
ROWSP = 104
NCHUNKS = BAGS_PER_W // CHUNK_BAGS
SUB = 4
SROWS = SUB * ROWSP
NSUPER = NCHUNKS // SUB
NVEC = HID // 16

_mesh = plsc.VectorSubcoreMesh(core_axis_name="c", subcore_axis_name="s")


def _gather(tab_hbm, idx_ref, rows_ref, sem):
    return pltpu.make_async_copy(tab_hbm.at[idx_ref], rows_ref, sem)


def _accumulate(rows_ref, ost_v):
    for q in range(SUB):
        for bag in range(CHUNK_BAGS):
            def body(r, accs, _base=q * ROWSP + bag * L):
                row = _base + r
                return tuple(
                    accs[k] + rows_ref[row, pl.ds(16 * k, 16)]
                    for k in range(NVEC)
                )

            zero = jnp.zeros((16,), jnp.float32)
            accs = lax.fori_loop(0, L, body, (zero,) * NVEC, unroll=10)
            slot = q * CHUNK_BAGS + bag
            for k in range(NVEC):
                ost_v[slot, pl.ds(16 * k, 16)] = accs[k]


@functools.partial(
    pl.kernel,
    mesh=_mesh,
    out_type=jax.ShapeDtypeStruct((B, HID), jnp.float32),
    scratch_types=[
        pltpu.VMEM((SROWS,), jnp.int32),
        pltpu.VMEM((SROWS,), jnp.int32),
        pltpu.VMEM((SROWS, HID), jnp.float32),
        pltpu.VMEM((SROWS, HID), jnp.float32),
        pltpu.VMEM((SUB * CHUNK_BAGS, HID), jnp.float32),
        pltpu.SemaphoreType.DMA,
        pltpu.SemaphoreType.DMA,
    ],
)
def _sc_pool(x_hbm, tab_hbm, out_hbm, idx0, idx1, rows0, rows1, ost_v,
             sem0, sem1):
    sid = lax.axis_index("s")
    wid = sid * NC + lax.axis_index("c")
    base = wid * BAGS_PER_W

    def stage_idx(gg, idx_ref):
        off = pl.multiple_of((wid * NSUPER + gg) * SROWS, 8)
        pltpu.sync_copy(x_hbm.at[pl.ds(off, SROWS)], idx_ref)

    bufs = ((idx0, rows0, sem0), (idx1, rows1, sem1))
    NBUF = len(bufs)
    for i, (idx, rows, sem) in enumerate(bufs):
        stage_idx(i, idx)
        _gather(tab_hbm, idx, rows, sem).start()

    @pl.loop(0, NSUPER, step=NBUF)
    def _(g):
        for step, (idx, rows, sem) in enumerate(bufs):
            gg = g + step
            _gather(tab_hbm, idx, rows, sem).wait()
            _accumulate(rows, ost_v)

            @pl.when(gg + NBUF < NSUPER)
            def _():
                stage_idx(gg + NBUF, idx)
                _gather(tab_hbm, idx, rows, sem).start()

            flush = SUB * CHUNK_BAGS
            off = pl.multiple_of(base + gg * flush, 8)
            pltpu.sync_copy(ost_v, out_hbm.at[pl.ds(off, flush)])


_MM_BLK = 2048


def _mm_body(et_ref, wt_ref, o_ref):
    o_ref[...] = lax.dot_general(
        et_ref[...], wt_ref[...],
        dimension_numbers=(((0,), (0,)), ((), ())),
        preferred_element_type=jnp.float32,
    )


def _tc_project_table(embT, Wt):
    return pl.pallas_call(
        _mm_body,
        grid=(pl.cdiv(VOCAB, _MM_BLK),),
        in_specs=[
            pl.BlockSpec((EMB, _MM_BLK), lambda i: (0, i)),
            pl.BlockSpec((EMB, HID), lambda i: (0, 0)),
        ],
        out_specs=pl.BlockSpec((_MM_BLK, HID), lambda i: (i, 0)),
        out_shape=jax.ShapeDtypeStruct((VOCAB, HID), jnp.float32),
    )(embT, Wt)


def _epi_body(s_ref, len_ref, b_ref, o_ref):
    o_ref[...] = s_ref[...] / len_ref[...] + b_ref[...]


def _tc_epilogue(sums, length_f, b2):
    return pl.pallas_call(
        _epi_body,
        grid=(1,),
        in_specs=[
            pl.BlockSpec((B, HID), lambda i: (0, 0)),
            pl.BlockSpec((B, 1), lambda i: (0, 0)),
            pl.BlockSpec((1, HID), lambda i: (0, 0)),
        ],
        out_specs=pl.BlockSpec((B, HID), lambda i: (0, 0)),
        out_shape=jax.ShapeDtypeStruct((B, HID), jnp.float32),
    )(sums, length_f, b2)


@jax.jit
def kernel(x, length, emb, W, b):
    P = _tc_project_table(emb.T, W.T)
    x3d = x.astype(jnp.int32).reshape(NW, NCHUNKS, ROWS)
    x3d = jnp.pad(x3d, ((0, 0), (0, 0), (0, ROWSP - ROWS)))
    sums = _sc_pool(x3d.reshape(NW * NSUPER * SROWS), P)
    length_f = length.astype(jnp.float32).reshape(B, 1)
    return _tc_epilogue(sums, length_f, b.reshape(1, HID))


# speedup vs baseline: 1.0346x; 1.0346x over previous
import functools

import jax
import jax.numpy as jnp
from jax import lax
from jax.experimental import pallas as pl
from jax.experimental.pallas import tpu as pltpu
from jax.experimental.pallas import tpu_sc as plsc

VOCAB = 100000
EMB = 300
HID = 128
B = 4096
L = 50

NC = 2
NS = 16
NW = NC * NS
BAGS_PER_W = B // NW
CHUNK_BAGS = 2
ROWS = CHUNK_BAGS * L
ROWSP = 104
NCHUNKS = BAGS_PER_W // CHUNK_BAGS
SUB = 4
SROWS = SUB * ROWSP
NSUPER = NCHUNKS // SUB
NVEC = HID // 16

_mesh = plsc.VectorSubcoreMesh(core_axis_name="c", subcore_axis_name="s")


def _gather(tab_hbm, idx_ref, rows_ref, sem):
    return pltpu.make_async_copy(tab_hbm.at[idx_ref], rows_ref, sem)


def _accumulate(rows_ref, ost_v):
    for q in range(SUB):
        for bag in range(CHUNK_BAGS):
            def body(r, accs, _base=q * ROWSP + bag * L):
                row = _base + r
                return tuple(
                    accs[k] + rows_ref[row, pl.ds(16 * k, 16)]
                    for k in range(NVEC)
                )

            zero = jnp.zeros((16,), jnp.float32)
            accs = lax.fori_loop(0, L, body, (zero,) * NVEC, unroll=10)
            slot = q * CHUNK_BAGS + bag
            for k in range(NVEC):
                ost_v[slot, pl.ds(16 * k, 16)] = accs[k]


@functools.partial(
    pl.kernel,
    mesh=_mesh,
    out_type=jax.ShapeDtypeStruct((B, HID), jnp.float32),
    scratch_types=[
        pltpu.VMEM((SROWS,), jnp.int32),
        pltpu.VMEM((SROWS,), jnp.int32),
        pltpu.VMEM((SROWS, HID), jnp.float32),
        pltpu.VMEM((SROWS, HID), jnp.float32),
        pltpu.VMEM((SUB * CHUNK_BAGS, HID), jnp.float32),
        pltpu.SemaphoreType.DMA,
        pltpu.SemaphoreType.DMA,
    ],
)
def _sc_pool(x_hbm, tab_hbm, out_hbm, idx0, idx1, rows0, rows1, ost_v,
             sem0, sem1):
    sid = lax.axis_index("s")
    wid = sid * NC + lax.axis_index("c")
    base = wid * BAGS_PER_W

    def stage_idx(gg, idx_ref):
        off = pl.multiple_of((wid * NSUPER + gg) * SROWS, 8)
        pltpu.sync_copy(x_hbm.at[pl.ds(off, SROWS)], idx_ref)

    bufs = ((idx0, rows0, sem0), (idx1, rows1, sem1))
    NBUF = len(bufs)
    for i, (idx, rows, sem) in enumerate(bufs):
        stage_idx(i, idx)
        _gather(tab_hbm, idx, rows, sem).start()

    @pl.loop(0, NSUPER, step=NBUF)
    def _(g):
        for step, (idx, rows, sem) in enumerate(bufs):
            gg = g + step
            _gather(tab_hbm, idx, rows, sem).wait()
            _accumulate(rows, ost_v)

            @pl.when(gg + NBUF < NSUPER)
            def _():
                stage_idx(gg + NBUF, idx)
                _gather(tab_hbm, idx, rows, sem).start()

            flush = SUB * CHUNK_BAGS
            off = pl.multiple_of(base + gg * flush, 8)
            pltpu.sync_copy(ost_v, out_hbm.at[pl.ds(off, flush)])


_MM_BLK = 4096


def _mm_body(et_ref, wt_ref, o_ref):
    o_ref[...] = lax.dot_general(
        et_ref[...], wt_ref[...],
        dimension_numbers=(((0,), (0,)), ((), ())),
        preferred_element_type=jnp.float32,
    )


def _tc_project_table(embT, Wt):
    return pl.pallas_call(
        _mm_body,
        grid=(pl.cdiv(VOCAB, _MM_BLK),),
        in_specs=[
            pl.BlockSpec((EMB, _MM_BLK), lambda i: (0, i)),
            pl.BlockSpec((EMB, HID), lambda i: (0, 0)),
        ],
        out_specs=pl.BlockSpec((_MM_BLK, HID), lambda i: (i, 0)),
        out_shape=jax.ShapeDtypeStruct((VOCAB, HID), jnp.float32),
    )(embT, Wt)


def _epi_body(s_ref, len_ref, b_ref, o_ref):
    o_ref[...] = s_ref[...] / len_ref[...] + b_ref[...]


def _tc_epilogue(sums, length_f, b2):
    return pl.pallas_call(
        _epi_body,
        grid=(1,),
        in_specs=[
            pl.BlockSpec((B, HID), lambda i: (0, 0)),
            pl.BlockSpec((B, 1), lambda i: (0, 0)),
            pl.BlockSpec((1, HID), lambda i: (0, 0)),
        ],
        out_specs=pl.BlockSpec((B, HID), lambda i: (0, 0)),
        out_shape=jax.ShapeDtypeStruct((B, HID), jnp.float32),
    )(sums, length_f, b2)


@jax.jit
def kernel(x, length, emb, W, b):
    P = _tc_project_table(emb.T, W.T)
    x3d = x.astype(jnp.int32).reshape(NW, NCHUNKS, ROWS)
    x3d = jnp.pad(x3d, ((0, 0), (0, 0), (0, ROWSP - ROWS)))
    sums = _sc_pool(x3d.reshape(NW * NSUPER * SROWS), P)
    length_f = length.astype(jnp.float32).reshape(B, 1)
    return _tc_epilogue(sums, length_f, b.reshape(1, HID))
